# baseline (speedup 1.0000x reference)
import functools

import jax
import jax.numpy as jnp
from jax import lax
from jax.experimental import pallas as pl
from jax.experimental.pallas import tpu as pltpu
from jax.experimental.pallas import tpu_sc as plsc

_N = 10000
_E = 320000
_D = 128
_R = 8

_NC = 2
_NS = 16
_NW = _NC * _NS
_CH = 128
_NCHUNK = 80
_EPAD = _NW * _NCHUNK * _CH
_NPAD = 10112
_RPT = _NPAD // _NS

_BLK = 400
_GRID = _N // _BLK


def _l0_body(x_ref, w_ref, wself_ref, xw_ref, self_ref):
    x = x_ref[...]
    for r in range(_R):
        xw_ref[:, r, :] = jnp.dot(x, w_ref[r],
                                  preferred_element_type=jnp.float32)
    self_ref[...] = jnp.dot(x, wself_ref[...],
                            preferred_element_type=jnp.float32)


def _flat_body(src_ref, typ_ref, dst_ref, gidx_ref, didx_ref):
    npad_rows = _EPAD // 128 - _E // 128
    pad = jnp.zeros((npad_rows, 128), jnp.int32)
    gidx_ref[...] = jnp.concatenate(
        [src_ref[...] * _R + typ_ref[...], pad], axis=0)
    didx_ref[...] = jnp.concatenate(
        [dst_ref[...], pad + _N], axis=0)


def _tc_flat_idx(src2d, typ2d, dst2d):
    return pl.pallas_call(
        _flat_body,
        out_shape=[
            jax.ShapeDtypeStruct((_EPAD // 128, 128), jnp.int32),
            jax.ShapeDtypeStruct((_EPAD // 128, 128), jnp.int32),
        ],
    )(src2d, typ2d, dst2d)


def _mid_body(x_ref, self0_ref, parts_ref, b0_ref, w_ref, wself_ref,
              xw_ref, self_ref):
    h = x_ref[...] + self0_ref[...] + parts_ref[0] + parts_ref[1]
    h = jnp.maximum(h, 0.0) + b0_ref[...]
    self_ref[...] = jnp.dot(h, wself_ref[...],
                            preferred_element_type=jnp.float32)
    for r in range(_R):
        xw_ref[:, r, :] = jnp.dot(h, w_ref[r],
                                  preferred_element_type=jnp.float32)


def _fin_body(self1_ref, parts_ref, b1_ref, o_ref):
    o_ref[...] = self1_ref[...] + parts_ref[0] + parts_ref[1] + b1_ref[...]


def _tc_layer0(x, W, wself):
    return pl.pallas_call(
        _l0_body,
        grid=(_GRID,),
        in_specs=[
            pl.BlockSpec((_BLK, _D), lambda i: (i, 0)),
            pl.BlockSpec((_R, _D, _D), lambda i: (0, 0, 0)),
            pl.BlockSpec((_D, _D), lambda i: (0, 0)),
        ],
        out_specs=[
            pl.BlockSpec((_BLK, _R, _D), lambda i: (i, 0, 0)),
            pl.BlockSpec((_BLK, _D), lambda i: (i, 0)),
        ],
        out_shape=[
            jax.ShapeDtypeStruct((_N, _R, _D), jnp.float32),
            jax.ShapeDtypeStruct((_N, _D), jnp.float32),
        ],
    )(x, W, wself)


def _tc_mid(x, self0, parts, b0row, W, wself):
    return pl.pallas_call(
        _mid_body,
        grid=(_GRID,),
        in_specs=[
            pl.BlockSpec((_BLK, _D), lambda i: (i, 0)),
            pl.BlockSpec((_BLK, _D), lambda i: (i, 0)),
            pl.BlockSpec((_NC, _BLK, _D), lambda i: (0, i, 0)),
            pl.BlockSpec((1, _D), lambda i: (0, 0)),
            pl.BlockSpec((_R, _D, _D), lambda i: (0, 0, 0)),
            pl.BlockSpec((_D, _D), lambda i: (0, 0)),
        ],
        out_specs=[
            pl.BlockSpec((_BLK, _R, _D), lambda i: (i, 0, 0)),
            pl.BlockSpec((_BLK, _D), lambda i: (i, 0)),
        ],
        out_shape=[
            jax.ShapeDtypeStruct((_N, _R, _D), jnp.float32),
            jax.ShapeDtypeStruct((_N, _D), jnp.float32),
        ],
    )(x, self0, parts, b0row, W, wself)


def _tc_final(self1, parts, b1row):
    return pl.pallas_call(
        _fin_body,
        grid=(_GRID,),
        in_specs=[
            pl.BlockSpec((_BLK, _D), lambda i: (i, 0)),
            pl.BlockSpec((_NC, _BLK, _D), lambda i: (0, i, 0)),
            pl.BlockSpec((1, _D), lambda i: (0, 0)),
        ],
        out_specs=pl.BlockSpec((_BLK, _D), lambda i: (i, 0)),
        out_shape=jax.ShapeDtypeStruct((_N, _D), jnp.float32),
    )(self1, parts, b1row)


def _make_sc_kernel():
    mesh = plsc.VectorSubcoreMesh(core_axis_name="c", subcore_axis_name="s")

    def body(table, gidx, didx, zinit, out, gidx_v, didx_v, rows, agg_s):
        c = lax.axis_index("c")
        s = lax.axis_index("s")
        wid = s * _NC + c
        pltpu.sync_copy(gidx.at[wid], gidx_v)
        pltpu.sync_copy(didx.at[wid], didx_v)
        pltpu.sync_copy(zinit.at[pl.ds(s * _RPT, _RPT)],
                        agg_s.at[pl.ds(s * _RPT, _RPT)])
        plsc.subcore_barrier()

        @pl.loop(0, _NCHUNK)
        def _chunk(j):
            pltpu.sync_copy(table.at[gidx_v.at[j]], rows)
            pltpu.sync_copy(rows, agg_s.at[didx_v.at[j]], add=True)

        plsc.subcore_barrier()
        pltpu.sync_copy(agg_s.at[pl.ds(s * _RPT, _RPT)],
                        out.at[c, pl.ds(s * _RPT, _RPT)])

    return pl.kernel(
        body,
        out_type=jax.ShapeDtypeStruct((_NC, _NPAD, _D), jnp.float32),
        mesh=mesh,
        scratch_types=[
            pltpu.VMEM((_NCHUNK, _CH), jnp.int32),
            pltpu.VMEM((_NCHUNK, _CH), jnp.int32),
            pltpu.VMEM((_CH, _D), jnp.float32),
            pltpu.VMEM_SHARED((_NPAD, _D), jnp.float32),
        ],
    )


@functools.cache
def _sc_kernel_cached():
    return _make_sc_kernel()


def _sc_gather_scatter(table, gidx, didx, zinit):
    return _sc_kernel_cached()(table, gidx, didx, zinit)


def kernel(x, edge_index, edge_type, W0, self_w0, b0, W1, self_w1, b1):
    src2d = edge_index[0].reshape(_E // 128, 128)
    typ2d = edge_type.reshape(_E // 128, 128)
    b0row = b0.reshape(1, _D)
    b1row = b1.reshape(1, _D)
    zinit = jnp.zeros((_NPAD, _D), jnp.float32)

    xw0, self0 = _tc_layer0(x, W0, self_w0)
    gidx2d, didx2d = _tc_flat_idx(src2d, typ2d,
                                  edge_index[1].reshape(_E // 128, 128))
    gidx = gidx2d.reshape(_NW, _NCHUNK, _CH)
    didx = didx2d.reshape(_NW, _NCHUNK, _CH)

    parts0 = _sc_gather_scatter(xw0.reshape(_N * _R, _D), gidx, didx, zinit)

    xw1, self1 = _tc_mid(x, self0, parts0, b0row, W1, self_w1)
    parts1 = _sc_gather_scatter(xw1.reshape(_N * _R, _D), gidx, didx, zinit)

    return _tc_final(self1, parts1, b1row)


# speedup vs baseline: 1.0001x; 1.0001x over previous
import functools

import jax
import jax.numpy as jnp
from jax import lax
from jax.experimental import pallas as pl
from jax.experimental.pallas import tpu as pltpu
from jax.experimental.pallas import tpu_sc as plsc

_N = 10000
_E = 320000
_D = 128
_R = 8

_NC = 2
_NS = 16
_NW = _NC * _NS
_CH = 128
_NCHUNK = 80
_EPAD = _NW * _NCHUNK * _CH
_NPAD = 10112
_RPT = _NPAD // _NS

_BLK = 400
_GRID = _N // _BLK


def _l0_body(x_ref, w_ref, wself_ref, xw_ref, self_ref):
    x = x_ref[...]
    for r in range(_R):
        xw_ref[:, r, :] = jnp.dot(x, w_ref[r],
                                  preferred_element_type=jnp.float32)
    self_ref[...] = jnp.dot(x, wself_ref[...],
                            preferred_element_type=jnp.float32)


def _flat_body(src_ref, typ_ref, dst_ref, gidx_ref, didx_ref):
    npad_rows = _EPAD // 128 - _E // 128
    pad_dst = _N + lax.broadcasted_iota(
        jnp.int32, (npad_rows, 128), 1) % (_NPAD - _N)
    gidx_ref[...] = jnp.concatenate(
        [src_ref[...] * _R + typ_ref[...],
         jnp.zeros((npad_rows, 128), jnp.int32)], axis=0)
    didx_ref[...] = jnp.concatenate([dst_ref[...], pad_dst], axis=0)


def _tc_flat_idx(src2d, typ2d, dst2d):
    return pl.pallas_call(
        _flat_body,
        out_shape=[
            jax.ShapeDtypeStruct((_EPAD // 128, 128), jnp.int32),
            jax.ShapeDtypeStruct((_EPAD // 128, 128), jnp.int32),
        ],
    )(src2d, typ2d, dst2d)


def _mid_body(x_ref, self0_ref, parts_ref, b0_ref, w_ref, wself_ref,
              xw_ref, self_ref):
    h = x_ref[...] + self0_ref[...] + parts_ref[0] + parts_ref[1]
    h = jnp.maximum(h, 0.0) + b0_ref[...]
    self_ref[...] = jnp.dot(h, wself_ref[...],
                            preferred_element_type=jnp.float32)
    for r in range(_R):
        xw_ref[:, r, :] = jnp.dot(h, w_ref[r],
                                  preferred_element_type=jnp.float32)


def _fin_body(self1_ref, parts_ref, b1_ref, o_ref):
    o_ref[...] = self1_ref[...] + parts_ref[0] + parts_ref[1] + b1_ref[...]


def _tc_layer0(x, W, wself):
    return pl.pallas_call(
        _l0_body,
        grid=(_GRID,),
        in_specs=[
            pl.BlockSpec((_BLK, _D), lambda i: (i, 0)),
            pl.BlockSpec((_R, _D, _D), lambda i: (0, 0, 0)),
            pl.BlockSpec((_D, _D), lambda i: (0, 0)),
        ],
        out_specs=[
            pl.BlockSpec((_BLK, _R, _D), lambda i: (i, 0, 0)),
            pl.BlockSpec((_BLK, _D), lambda i: (i, 0)),
        ],
        out_shape=[
            jax.ShapeDtypeStruct((_N, _R, _D), jnp.float32),
            jax.ShapeDtypeStruct((_N, _D), jnp.float32),
        ],
    )(x, W, wself)


def _tc_mid(x, self0, parts, b0row, W, wself):
    return pl.pallas_call(
        _mid_body,
        grid=(_GRID,),
        in_specs=[
            pl.BlockSpec((_BLK, _D), lambda i: (i, 0)),
            pl.BlockSpec((_BLK, _D), lambda i: (i, 0)),
            pl.BlockSpec((_NC, _BLK, _D), lambda i: (0, i, 0)),
            pl.BlockSpec((1, _D), lambda i: (0, 0)),
            pl.BlockSpec((_R, _D, _D), lambda i: (0, 0, 0)),
            pl.BlockSpec((_D, _D), lambda i: (0, 0)),
        ],
        out_specs=[
            pl.BlockSpec((_BLK, _R, _D), lambda i: (i, 0, 0)),
            pl.BlockSpec((_BLK, _D), lambda i: (i, 0)),
        ],
        out_shape=[
            jax.ShapeDtypeStruct((_N, _R, _D), jnp.float32),
            jax.ShapeDtypeStruct((_N, _D), jnp.float32),
        ],
    )(x, self0, parts, b0row, W, wself)


def _tc_final(self1, parts, b1row):
    return pl.pallas_call(
        _fin_body,
        grid=(_GRID,),
        in_specs=[
            pl.BlockSpec((_BLK, _D), lambda i: (i, 0)),
            pl.BlockSpec((_NC, _BLK, _D), lambda i: (0, i, 0)),
            pl.BlockSpec((1, _D), lambda i: (0, 0)),
        ],
        out_specs=pl.BlockSpec((_BLK, _D), lambda i: (i, 0)),
        out_shape=jax.ShapeDtypeStruct((_N, _D), jnp.float32),
    )(self1, parts, b1row)


def _make_sc_kernel():
    mesh = plsc.VectorSubcoreMesh(core_axis_name="c", subcore_axis_name="s")

    def body(table, gidx, didx, zinit, out, gidx_v, didx_v, rows, agg_s):
        c = lax.axis_index("c")
        s = lax.axis_index("s")
        wid = s * _NC + c
        pltpu.sync_copy(gidx.at[wid], gidx_v)
        pltpu.sync_copy(didx.at[wid], didx_v)
        pltpu.sync_copy(zinit.at[pl.ds(s * _RPT, _RPT)],
                        agg_s.at[pl.ds(s * _RPT, _RPT)])
        plsc.subcore_barrier()

        @pl.loop(0, _NCHUNK)
        def _chunk(j):
            pltpu.sync_copy(table.at[gidx_v.at[j]], rows)
            pltpu.sync_copy(rows, agg_s.at[didx_v.at[j]], add=True)

        plsc.subcore_barrier()
        pltpu.sync_copy(agg_s.at[pl.ds(s * _RPT, _RPT)],
                        out.at[c, pl.ds(s * _RPT, _RPT)])

    return pl.kernel(
        body,
        out_type=jax.ShapeDtypeStruct((_NC, _NPAD, _D), jnp.float32),
        mesh=mesh,
        scratch_types=[
            pltpu.VMEM((_NCHUNK, _CH), jnp.int32),
            pltpu.VMEM((_NCHUNK, _CH), jnp.int32),
            pltpu.VMEM((_CH, _D), jnp.float32),
            pltpu.VMEM_SHARED((_NPAD, _D), jnp.float32),
        ],
    )


@functools.cache
def _sc_kernel_cached():
    return _make_sc_kernel()


def _sc_gather_scatter(table, gidx, didx, zinit):
    return _sc_kernel_cached()(table, gidx, didx, zinit)


def kernel(x, edge_index, edge_type, W0, self_w0, b0, W1, self_w1, b1):
    src2d = edge_index[0].reshape(_E // 128, 128)
    typ2d = edge_type.reshape(_E // 128, 128)
    b0row = b0.reshape(1, _D)
    b1row = b1.reshape(1, _D)
    zinit = jnp.zeros((_NPAD, _D), jnp.float32)

    xw0, self0 = _tc_layer0(x, W0, self_w0)
    gidx2d, didx2d = _tc_flat_idx(src2d, typ2d,
                                  edge_index[1].reshape(_E // 128, 128))
    gidx = gidx2d.reshape(_NW, _NCHUNK, _CH)
    didx = didx2d.reshape(_NW, _NCHUNK, _CH)

    parts0 = _sc_gather_scatter(xw0.reshape(_N * _R, _D), gidx, didx, zinit)

    xw1, self1 = _tc_mid(x, self0, parts0, b0row, W1, self_w1)
    parts1 = _sc_gather_scatter(xw1.reshape(_N * _R, _D), gidx, didx, zinit)

    return _tc_final(self1, parts1, b1row)


# speedup vs baseline: 2.2796x; 2.2793x over previous
import functools

import jax
import jax.numpy as jnp
from jax import lax
from jax.experimental import pallas as pl
from jax.experimental.pallas import tpu as pltpu
from jax.experimental.pallas import tpu_sc as plsc

_N = 10000
_E = 320000
_D = 128
_R = 8

_NC = 2
_NS = 16
_NW = _NC * _NS
_CH = 100
_NCHUNK = 100
_NPAD = 10240
_RPT = _NPAD // _NS

_BLK = 400
_GRID = _N // _BLK


def _l0_body(x_ref, w_ref, wself_ref, xw_ref, self_ref):
    x = x_ref[...]
    for r in range(_R):
        xw_ref[:, r, :] = jnp.dot(x, w_ref[r],
                                  preferred_element_type=jnp.float32)
    self_ref[...] = jnp.dot(x, wself_ref[...],
                            preferred_element_type=jnp.float32)


def _flat_body(src_ref, typ_ref, gidx_ref):
    gidx_ref[...] = src_ref[...] * _R + typ_ref[...]


def _tc_flat_idx(src2d, typ2d):
    return pl.pallas_call(
        _flat_body,
        out_shape=jax.ShapeDtypeStruct((_E // 128, 128), jnp.int32),
    )(src2d, typ2d)


def _mid_body(x_ref, self0_ref, parts_ref, b0_ref, w_ref, wself_ref,
              xw_ref, self_ref):
    h = x_ref[...] + self0_ref[...] + parts_ref[0] + parts_ref[1]
    h = jnp.maximum(h, 0.0) + b0_ref[...]
    self_ref[...] = jnp.dot(h, wself_ref[...],
                            preferred_element_type=jnp.float32)
    for r in range(_R):
        xw_ref[:, r, :] = jnp.dot(h, w_ref[r],
                                  preferred_element_type=jnp.float32)


def _fin_body(self1_ref, parts_ref, b1_ref, o_ref):
    o_ref[...] = self1_ref[...] + parts_ref[0] + parts_ref[1] + b1_ref[...]


def _tc_layer0(x, W, wself):
    return pl.pallas_call(
        _l0_body,
        grid=(_GRID,),
        in_specs=[
            pl.BlockSpec((_BLK, _D), lambda i: (i, 0)),
            pl.BlockSpec((_R, _D, _D), lambda i: (0, 0, 0)),
            pl.BlockSpec((_D, _D), lambda i: (0, 0)),
        ],
        out_specs=[
            pl.BlockSpec((_BLK, _R, _D), lambda i: (i, 0, 0)),
            pl.BlockSpec((_BLK, _D), lambda i: (i, 0)),
        ],
        out_shape=[
            jax.ShapeDtypeStruct((_N, _R, _D), jnp.float32),
            jax.ShapeDtypeStruct((_N, _D), jnp.float32),
        ],
    )(x, W, wself)


def _tc_mid(x, self0, parts, b0row, W, wself):
    return pl.pallas_call(
        _mid_body,
        grid=(_GRID,),
        in_specs=[
            pl.BlockSpec((_BLK, _D), lambda i: (i, 0)),
            pl.BlockSpec((_BLK, _D), lambda i: (i, 0)),
            pl.BlockSpec((_NC, _BLK, _D), lambda i: (0, i, 0)),
            pl.BlockSpec((1, _D), lambda i: (0, 0)),
            pl.BlockSpec((_R, _D, _D), lambda i: (0, 0, 0)),
            pl.BlockSpec((_D, _D), lambda i: (0, 0)),
        ],
        out_specs=[
            pl.BlockSpec((_BLK, _R, _D), lambda i: (i, 0, 0)),
            pl.BlockSpec((_BLK, _D), lambda i: (i, 0)),
        ],
        out_shape=[
            jax.ShapeDtypeStruct((_N, _R, _D), jnp.float32),
            jax.ShapeDtypeStruct((_N, _D), jnp.float32),
        ],
    )(x, self0, parts, b0row, W, wself)


def _tc_final(self1, parts, b1row):
    return pl.pallas_call(
        _fin_body,
        grid=(_GRID,),
        in_specs=[
            pl.BlockSpec((_BLK, _D), lambda i: (i, 0)),
            pl.BlockSpec((_NC, _BLK, _D), lambda i: (0, i, 0)),
            pl.BlockSpec((1, _D), lambda i: (0, 0)),
        ],
        out_specs=pl.BlockSpec((_BLK, _D), lambda i: (i, 0)),
        out_shape=jax.ShapeDtypeStruct((_N, _D), jnp.float32),
    )(self1, parts, b1row)


def _make_sc_kernel():
    mesh = plsc.VectorSubcoreMesh(core_axis_name="c", subcore_axis_name="s")

    def body(table, gidx, didx, zinit, out, gidx_v, didx_v, rows, agg_s):
        c = lax.axis_index("c")
        s = lax.axis_index("s")
        wid = s * _NC + c
        pltpu.sync_copy(gidx.at[wid], gidx_v)
        pltpu.sync_copy(didx.at[wid], didx_v)
        pltpu.sync_copy(zinit.at[pl.ds(s * _RPT, _RPT)],
                        agg_s.at[pl.ds(s * _RPT, _RPT)])
        plsc.subcore_barrier()

        @pl.loop(0, _NCHUNK)
        def _chunk(j):
            pltpu.sync_copy(table.at[gidx_v.at[j]], rows)
            pltpu.sync_copy(rows, agg_s.at[didx_v.at[j]], add=True)

        plsc.subcore_barrier()
        pltpu.sync_copy(agg_s.at[pl.ds(s * _RPT, _RPT)],
                        out.at[c, pl.ds(s * _RPT, _RPT)])

    return pl.kernel(
        body,
        out_type=jax.ShapeDtypeStruct((_NC, _NPAD, _D), jnp.float32),
        mesh=mesh,
        scratch_types=[
            pltpu.VMEM((_NCHUNK, _CH), jnp.int32),
            pltpu.VMEM((_NCHUNK, _CH), jnp.int32),
            pltpu.VMEM((_CH, _D), jnp.float32),
            pltpu.VMEM_SHARED((_NPAD, _D), jnp.float32),
        ],
    )


@functools.cache
def _sc_kernel_cached():
    return _make_sc_kernel()


def _sc_gather_scatter(table, gidx, didx, zinit):
    return _sc_kernel_cached()(table, gidx, didx, zinit)


def kernel(x, edge_index, edge_type, W0, self_w0, b0, W1, self_w1, b1):
    src2d = edge_index[0].reshape(_E // 128, 128)
    typ2d = edge_type.reshape(_E // 128, 128)
    b0row = b0.reshape(1, _D)
    b1row = b1.reshape(1, _D)
    zinit = jnp.zeros((_NPAD, _D), jnp.float32)

    xw0, self0 = _tc_layer0(x, W0, self_w0)
    gidx = _tc_flat_idx(src2d, typ2d).reshape(_NW, _NCHUNK, _CH)
    didx = edge_index[1].reshape(_NW, _NCHUNK, _CH)

    parts0 = _sc_gather_scatter(xw0.reshape(_N * _R, _D), gidx, didx, zinit)

    xw1, self1 = _tc_mid(x, self0, parts0, b0row, W1, self_w1)
    parts1 = _sc_gather_scatter(xw1.reshape(_N * _R, _D), gidx, didx, zinit)

    return _tc_final(self1, parts1, b1row)


# speedup vs baseline: 2.4015x; 1.0535x over previous
import functools

import jax
import jax.numpy as jnp
from jax import lax
from jax.experimental import pallas as pl
from jax.experimental.pallas import tpu as pltpu
from jax.experimental.pallas import tpu_sc as plsc

_N = 10000
_E = 320000
_D = 128
_R = 8

_NC = 2
_NS = 16
_NW = _NC * _NS
_CH = 125
_NCHUNK = 80
_NPAD = 10240
_RPT = _NPAD // _NS

_BLK = 400
_GRID = _N // _BLK


def _l0_body(x_ref, w_ref, wself_ref, xw_ref, self_ref):
    x = x_ref[...]
    for r in range(_R):
        xw_ref[:, r, :] = jnp.dot(x, w_ref[r],
                                  preferred_element_type=jnp.float32)
    self_ref[...] = jnp.dot(x, wself_ref[...],
                            preferred_element_type=jnp.float32)


def _flat_body(src_ref, typ_ref, gidx_ref):
    gidx_ref[...] = src_ref[...] * _R + typ_ref[...]


def _tc_flat_idx(src2d, typ2d):
    return pl.pallas_call(
        _flat_body,
        out_shape=jax.ShapeDtypeStruct((_E // 128, 128), jnp.int32),
    )(src2d, typ2d)


def _mid_body(x_ref, self0_ref, parts_ref, b0_ref, w_ref, wself_ref,
              xw_ref, self_ref):
    h = x_ref[...] + self0_ref[...] + parts_ref[0] + parts_ref[1]
    h = jnp.maximum(h, 0.0) + b0_ref[...]
    self_ref[...] = jnp.dot(h, wself_ref[...],
                            preferred_element_type=jnp.float32)
    for r in range(_R):
        xw_ref[:, r, :] = jnp.dot(h, w_ref[r],
                                  preferred_element_type=jnp.float32)


def _fin_body(self1_ref, parts_ref, b1_ref, o_ref):
    o_ref[...] = self1_ref[...] + parts_ref[0] + parts_ref[1] + b1_ref[...]


def _tc_layer0(x, W, wself):
    return pl.pallas_call(
        _l0_body,
        grid=(_GRID,),
        in_specs=[
            pl.BlockSpec((_BLK, _D), lambda i: (i, 0)),
            pl.BlockSpec((_R, _D, _D), lambda i: (0, 0, 0)),
            pl.BlockSpec((_D, _D), lambda i: (0, 0)),
        ],
        out_specs=[
            pl.BlockSpec((_BLK, _R, _D), lambda i: (i, 0, 0)),
            pl.BlockSpec((_BLK, _D), lambda i: (i, 0)),
        ],
        out_shape=[
            jax.ShapeDtypeStruct((_N, _R, _D), jnp.float32),
            jax.ShapeDtypeStruct((_N, _D), jnp.float32),
        ],
    )(x, W, wself)


def _tc_mid(x, self0, parts, b0row, W, wself):
    return pl.pallas_call(
        _mid_body,
        grid=(_GRID,),
        in_specs=[
            pl.BlockSpec((_BLK, _D), lambda i: (i, 0)),
            pl.BlockSpec((_BLK, _D), lambda i: (i, 0)),
            pl.BlockSpec((_NC, _BLK, _D), lambda i: (0, i, 0)),
            pl.BlockSpec((1, _D), lambda i: (0, 0)),
            pl.BlockSpec((_R, _D, _D), lambda i: (0, 0, 0)),
            pl.BlockSpec((_D, _D), lambda i: (0, 0)),
        ],
        out_specs=[
            pl.BlockSpec((_BLK, _R, _D), lambda i: (i, 0, 0)),
            pl.BlockSpec((_BLK, _D), lambda i: (i, 0)),
        ],
        out_shape=[
            jax.ShapeDtypeStruct((_N, _R, _D), jnp.float32),
            jax.ShapeDtypeStruct((_N, _D), jnp.float32),
        ],
    )(x, self0, parts, b0row, W, wself)


def _tc_final(self1, parts, b1row):
    return pl.pallas_call(
        _fin_body,
        grid=(_GRID,),
        in_specs=[
            pl.BlockSpec((_BLK, _D), lambda i: (i, 0)),
            pl.BlockSpec((_NC, _BLK, _D), lambda i: (0, i, 0)),
            pl.BlockSpec((1, _D), lambda i: (0, 0)),
        ],
        out_specs=pl.BlockSpec((_BLK, _D), lambda i: (i, 0)),
        out_shape=jax.ShapeDtypeStruct((_N, _D), jnp.float32),
    )(self1, parts, b1row)


def _make_sc_kernel():
    mesh = plsc.VectorSubcoreMesh(core_axis_name="c", subcore_axis_name="s")

    def body(table, gidx, didx, zinit, out, gidx_v, didx_v, rows, agg_s):
        c = lax.axis_index("c")
        s = lax.axis_index("s")
        wid = s * _NC + c
        pltpu.sync_copy(gidx.at[wid], gidx_v)
        pltpu.sync_copy(didx.at[wid], didx_v)
        pltpu.sync_copy(zinit.at[pl.ds(s * _RPT, _RPT)],
                        agg_s.at[pl.ds(s * _RPT, _RPT)])
        plsc.subcore_barrier()

        @pl.loop(0, _NCHUNK)
        def _chunk(j):
            pltpu.sync_copy(table.at[gidx_v.at[j]], rows)
            pltpu.sync_copy(rows, agg_s.at[didx_v.at[j]], add=True)

        plsc.subcore_barrier()
        pltpu.sync_copy(agg_s.at[pl.ds(s * _RPT, _RPT)],
                        out.at[c, pl.ds(s * _RPT, _RPT)])

    return pl.kernel(
        body,
        out_type=jax.ShapeDtypeStruct((_NC, _NPAD, _D), jnp.float32),
        mesh=mesh,
        scratch_types=[
            pltpu.VMEM((_NCHUNK, _CH), jnp.int32),
            pltpu.VMEM((_NCHUNK, _CH), jnp.int32),
            pltpu.VMEM((_CH, _D), jnp.float32),
            pltpu.VMEM_SHARED((_NPAD, _D), jnp.float32),
        ],
    )


@functools.cache
def _sc_kernel_cached():
    return _make_sc_kernel()


def _sc_gather_scatter(table, gidx, didx, zinit):
    return _sc_kernel_cached()(table, gidx, didx, zinit)


def kernel(x, edge_index, edge_type, W0, self_w0, b0, W1, self_w1, b1):
    src2d = edge_index[0].reshape(_E // 128, 128)
    typ2d = edge_type.reshape(_E // 128, 128)
    b0row = b0.reshape(1, _D)
    b1row = b1.reshape(1, _D)
    zinit = jnp.zeros((_NPAD, _D), jnp.float32)

    xw0, self0 = _tc_layer0(x, W0, self_w0)
    gidx = _tc_flat_idx(src2d, typ2d).reshape(_NW, _NCHUNK, _CH)
    didx = edge_index[1].reshape(_NW, _NCHUNK, _CH)

    parts0 = _sc_gather_scatter(xw0.reshape(_N * _R, _D), gidx, didx, zinit)

    xw1, self1 = _tc_mid(x, self0, parts0, b0row, W1, self_w1)
    parts1 = _sc_gather_scatter(xw1.reshape(_N * _R, _D), gidx, didx, zinit)

    return _tc_final(self1, parts1, b1row)


# speedup vs baseline: 3.2963x; 1.3726x over previous
import functools

import jax
import jax.numpy as jnp
from jax import lax
from jax.experimental import pallas as pl
from jax.experimental.pallas import tpu as pltpu
from jax.experimental.pallas import tpu_sc as plsc

_N = 10000
_E = 320000
_D = 128
_R = 8

_NC = 2
_NS = 16
_NW = _NC * _NS
_CH = 125
_NCHUNK = 80
_NPAD = 10240
_RPT = _NPAD // _NS

_BLK = 400
_GRID = _N // _BLK


def _l0_body(x_ref, w_ref, wself_ref, xw_ref, self_ref):
    x = x_ref[...]
    for r in range(_R):
        xw_ref[:, r, :] = jnp.dot(x, w_ref[r],
                                  preferred_element_type=jnp.float32)
    self_ref[...] = jnp.dot(x, wself_ref[...],
                            preferred_element_type=jnp.float32)


def _flat_body(src_ref, typ_ref, gidx_ref):
    gidx_ref[...] = src_ref[...] * _R + typ_ref[...]


def _tc_flat_idx(src2d, typ2d):
    return pl.pallas_call(
        _flat_body,
        out_shape=jax.ShapeDtypeStruct((_E // 128, 128), jnp.int32),
    )(src2d, typ2d)


def _mid_body(x_ref, self0_ref, parts_ref, b0_ref, w_ref, wself_ref,
              xw_ref, self_ref):
    h = x_ref[...] + self0_ref[...] + parts_ref[0] + parts_ref[1]
    h = jnp.maximum(h, 0.0) + b0_ref[...]
    self_ref[...] = jnp.dot(h, wself_ref[...],
                            preferred_element_type=jnp.float32)
    for r in range(_R):
        xw_ref[:, r, :] = jnp.dot(h, w_ref[r],
                                  preferred_element_type=jnp.float32)


def _fin_body(self1_ref, parts_ref, b1_ref, o_ref):
    o_ref[...] = self1_ref[...] + parts_ref[0] + parts_ref[1] + b1_ref[...]


def _tc_layer0(x, W, wself):
    return pl.pallas_call(
        _l0_body,
        grid=(_GRID,),
        in_specs=[
            pl.BlockSpec((_BLK, _D), lambda i: (i, 0)),
            pl.BlockSpec((_R, _D, _D), lambda i: (0, 0, 0)),
            pl.BlockSpec((_D, _D), lambda i: (0, 0)),
        ],
        out_specs=[
            pl.BlockSpec((_BLK, _R, _D), lambda i: (i, 0, 0)),
            pl.BlockSpec((_BLK, _D), lambda i: (i, 0)),
        ],
        out_shape=[
            jax.ShapeDtypeStruct((_N, _R, _D), jnp.float32),
            jax.ShapeDtypeStruct((_N, _D), jnp.float32),
        ],
    )(x, W, wself)


def _tc_mid(x, self0, parts, b0row, W, wself):
    return pl.pallas_call(
        _mid_body,
        grid=(_GRID,),
        in_specs=[
            pl.BlockSpec((_BLK, _D), lambda i: (i, 0)),
            pl.BlockSpec((_BLK, _D), lambda i: (i, 0)),
            pl.BlockSpec((_NC, _BLK, _D), lambda i: (0, i, 0)),
            pl.BlockSpec((1, _D), lambda i: (0, 0)),
            pl.BlockSpec((_R, _D, _D), lambda i: (0, 0, 0)),
            pl.BlockSpec((_D, _D), lambda i: (0, 0)),
        ],
        out_specs=[
            pl.BlockSpec((_BLK, _R, _D), lambda i: (i, 0, 0)),
            pl.BlockSpec((_BLK, _D), lambda i: (i, 0)),
        ],
        out_shape=[
            jax.ShapeDtypeStruct((_N, _R, _D), jnp.float32),
            jax.ShapeDtypeStruct((_N, _D), jnp.float32),
        ],
    )(x, self0, parts, b0row, W, wself)


def _tc_final(self1, parts, b1row):
    return pl.pallas_call(
        _fin_body,
        grid=(_GRID,),
        in_specs=[
            pl.BlockSpec((_BLK, _D), lambda i: (i, 0)),
            pl.BlockSpec((_NC, _BLK, _D), lambda i: (0, i, 0)),
            pl.BlockSpec((1, _D), lambda i: (0, 0)),
        ],
        out_specs=pl.BlockSpec((_BLK, _D), lambda i: (i, 0)),
        out_shape=jax.ShapeDtypeStruct((_N, _D), jnp.float32),
    )(self1, parts, b1row)


def _make_sc_kernel():
    mesh = plsc.VectorSubcoreMesh(core_axis_name="c", subcore_axis_name="s")

    def body(table, gidx, didx, zinit, out, gidx_v, didx_v, rows, agg_s,
             sem0, sem1):
        c = lax.axis_index("c")
        s = lax.axis_index("s")
        wid = s * _NC + c
        rows0 = rows.at[0]
        rows1 = rows.at[1]
        hc = _NCHUNK // 2
        pltpu.sync_copy(zinit.at[pl.ds(s * _RPT, _RPT)],
                        agg_s.at[pl.ds(s * _RPT, _RPT)])
        plsc.subcore_barrier()

        for h in range(2):
            pltpu.sync_copy(gidx.at[wid].at[pl.ds(h * hc, hc)], gidx_v)
            pltpu.sync_copy(didx.at[wid].at[pl.ds(h * hc, hc)], didx_v)
            pltpu.async_copy(table.at[gidx_v.at[0]], rows0, sem0)
            pltpu.async_copy(table.at[gidx_v.at[1]], rows1, sem1)

            @pl.loop(0, hc, step=2)
            def _chunk(j):
                pltpu.make_async_copy(table.at[gidx_v.at[j]], rows0,
                                      sem0).wait()
                pltpu.sync_copy(rows0, agg_s.at[didx_v.at[j]], add=True)

                @pl.when(j + 2 < hc)
                def _next0():
                    pltpu.async_copy(table.at[gidx_v.at[j + 2]], rows0, sem0)

                pltpu.make_async_copy(table.at[gidx_v.at[j + 1]], rows1,
                                      sem1).wait()
                pltpu.sync_copy(rows1, agg_s.at[didx_v.at[j + 1]], add=True)

                @pl.when(j + 3 < hc)
                def _next1():
                    pltpu.async_copy(table.at[gidx_v.at[j + 3]], rows1, sem1)

        plsc.subcore_barrier()
        pltpu.sync_copy(agg_s.at[pl.ds(s * _RPT, _RPT)],
                        out.at[c, pl.ds(s * _RPT, _RPT)])

    return pl.kernel(
        body,
        out_type=jax.ShapeDtypeStruct((_NC, _NPAD, _D), jnp.float32),
        mesh=mesh,
        scratch_types=[
            pltpu.VMEM((_NCHUNK // 2, _CH), jnp.int32),
            pltpu.VMEM((_NCHUNK // 2, _CH), jnp.int32),
            pltpu.VMEM((2, _CH, _D), jnp.float32),
            pltpu.VMEM_SHARED((_NPAD, _D), jnp.float32),
            pltpu.SemaphoreType.DMA,
            pltpu.SemaphoreType.DMA,
        ],
    )


@functools.cache
def _sc_kernel_cached():
    return _make_sc_kernel()


def _sc_gather_scatter(table, gidx, didx, zinit):
    return _sc_kernel_cached()(table, gidx, didx, zinit)


def kernel(x, edge_index, edge_type, W0, self_w0, b0, W1, self_w1, b1):
    src2d = edge_index[0].reshape(_E // 128, 128)
    typ2d = edge_type.reshape(_E // 128, 128)
    b0row = b0.reshape(1, _D)
    b1row = b1.reshape(1, _D)
    zinit = jnp.zeros((_NPAD, _D), jnp.float32)

    xw0, self0 = _tc_layer0(x, W0, self_w0)
    gidx = _tc_flat_idx(src2d, typ2d).reshape(_NW, _NCHUNK, _CH)
    didx = edge_index[1].reshape(_NW, _NCHUNK, _CH)

    parts0 = _sc_gather_scatter(xw0.reshape(_N * _R, _D), gidx, didx, zinit)

    xw1, self1 = _tc_mid(x, self0, parts0, b0row, W1, self_w1)
    parts1 = _sc_gather_scatter(xw1.reshape(_N * _R, _D), gidx, didx, zinit)

    return _tc_final(self1, parts1, b1row)
